# baseline (speedup 1.0000x reference)
import functools
import math

import jax
import jax.numpy as jnp
from jax import lax
from jax.experimental import pallas as pl
from jax.experimental.pallas import tpu as pltpu
from jax.experimental.pallas import tpu_sc as plsc

ALPHA = 0.5
BETA1 = math.log(2.0)
BETA2 = math.log(1.5)

NC = 2
NS = 16

N = 10000
D = 128
E = 160000
NPAD = 10240
EPAD = 163840
CHK = 16384
NCHK = EPAD // CHK
RPT = NPAD // NS
CPT = EPAD // NS // 128
FPT = 4

_mesh = plsc.VectorSubcoreMesh(
    core_axis_name="c", subcore_axis_name="s", num_cores=NC, num_subcores=NS)


@functools.partial(
    pl.kernel,
    out_type=jax.ShapeDtypeStruct((4, NPAD), jnp.float32),
    mesh=_mesh,
    scratch_types=[
        pltpu.VMEM((8, 128), jnp.int32),
        pltpu.VMEM((128,), jnp.float32),
        pltpu.VMEM((RPT,), jnp.float32),
        pltpu.VMEM_SHARED((NPAD,), jnp.float32),
        pltpu.VMEM_SHARED((NPAD,), jnp.float32),
    ],
)
def _sc_hist(hidx, deg_out, idxb, ones, zb, hist_s, hist_d):
    c = lax.axis_index("c")
    s = lax.axis_index("s")

    ov = jnp.ones((16,), jnp.float32)
    zv = jnp.zeros((16,), jnp.float32)

    def fill_ones(i, carry):
        ones[pl.ds(i * 16, 16)] = ov
        return carry
    lax.fori_loop(0, 8, fill_ones, 0)

    def fill_zb(i, carry):
        zb[pl.ds(i * 16, 16)] = zv
        return carry
    lax.fori_loop(0, RPT // 16, fill_zb, 0)

    pltpu.sync_copy(zb, hist_s.at[pl.ds(s * RPT, RPT)])
    pltpu.sync_copy(zb, hist_d.at[pl.ds(s * RPT, RPT)])
    plsc.subcore_barrier()

    base = s * CPT
    for j, hist in ((0, hist_s), (1, hist_d)):
        def grp(g, carry):
            pltpu.sync_copy(hidx.at[c, j, pl.ds(base + g * 8, 8), :], idxb)
            for k in range(8):
                pltpu.sync_copy(ones, hist.at[idxb.at[k]], add=True)
            return carry
        lax.fori_loop(0, CPT // 8, grp, 0)

    plsc.subcore_barrier()
    pltpu.sync_copy(hist_s.at[pl.ds(s * RPT, RPT)],
                    deg_out.at[2 * c, pl.ds(s * RPT, RPT)])
    pltpu.sync_copy(hist_d.at[pl.ds(s * RPT, RPT)],
                    deg_out.at[2 * c + 1, pl.ds(s * RPT, RPT)])


@functools.partial(
    pl.kernel,
    out_type=jax.ShapeDtypeStruct((2, D, NPAD), jnp.float32),
    mesh=_mesh,
    compiler_params=pltpu.CompilerParams(needs_layout_passes=False),
    scratch_types=[
        pltpu.VMEM((CHK,), jnp.int32),
        pltpu.VMEM((CHK,), jnp.int32),
        pltpu.VMEM((FPT * NPAD,), jnp.float32),
        pltpu.VMEM((FPT * NPAD,), jnp.float32),
    ],
)
def _sc_agg(tblT, eflat, out, srcb, dstb, tbl, acc):
    c = lax.axis_index("c")
    s = lax.axis_index("s")

    zv = jnp.zeros((16,), jnp.float32)

    for p in (0, 1):
        row0 = p * (NS * FPT) + s * FPT
        for j in range(FPT):
            pltpu.sync_copy(tblT.at[c, row0 + j, :],
                            tbl.at[pl.ds(j * NPAD, NPAD)])

        def zrow(i, carry):
            acc[pl.ds(i * 16, 16)] = zv
            return carry
        lax.fori_loop(0, FPT * NPAD // 16, zrow, 0)

        def chunk(g, carry):
            pltpu.sync_copy(eflat.at[c, 0, pl.ds(g * CHK, CHK)], srcb)
            pltpu.sync_copy(eflat.at[c, 1, pl.ds(g * CHK, CHK)], dstb)

            def ed(i, carry2):
                sv = [srcb[pl.ds(i * 128 + q * 16, 16)] for q in range(8)]
                dv = [dstb[pl.ds(i * 128 + q * 16, 16)] for q in range(8)]
                vs = [[plsc.load_gather(tbl, [sv[q] + (j * NPAD)])
                       for j in range(FPT)] for q in range(8)]
                for q in range(8):
                    for j in range(FPT):
                        plsc.addupdate_scatter(acc, [dv[q] + (j * NPAD)],
                                               vs[q][j])
                return carry2
            lax.fori_loop(0, CHK // 128, ed, 0)
            return carry
        lax.fori_loop(0, NCHK, chunk, 0)

        for j in range(FPT):
            pltpu.sync_copy(acc.at[pl.ds(j * NPAD, NPAD)],
                            out.at[c, row0 + j, :])


_R = 1024


def _norms(deg_blk):
    return lax.rsqrt(jnp.maximum(deg_blk, 1.0))


def _scale_body(deg_ref, xT_ref, xsT_ref):
    n = _norms(deg_ref[...])
    xb = xT_ref[...]
    for r in (0, 1):
        xsT_ref[r, :, :] = xb * n[2 * r:2 * r + 1, :]


def _tc_scale(deg, xT):
    g = NPAD // _R
    return pl.pallas_call(
        _scale_body,
        grid=(g,),
        in_specs=[pl.BlockSpec((4, _R), lambda i: (0, i)),
                  pl.BlockSpec((D, _R), lambda i: (0, i))],
        out_specs=pl.BlockSpec((2, D, _R), lambda i: (0, 0, i)),
        out_shape=jax.ShapeDtypeStruct((2, D, NPAD), jnp.float32),
    )(deg, xT)


def _combine1_body(agg_ref, xT_ref, deg_ref, w_ref, b_ref, hsT_ref):
    n = _norms(deg_ref[...])
    xb = xT_ref[...]
    acts = []
    for r in (0, 1):
        a = agg_ref[r] * n[2 * r + 1:2 * r + 2, :]
        f = (1.0 - ALPHA) * a + ALPHA * xb
        t = ((1.0 - BETA1) * f
             + BETA1 * jnp.dot(w_ref[r], f, preferred_element_type=jnp.float32)
             + b_ref[r])
        acts.append(jnp.where(t >= 0, t, 0.01 * t))
    h1 = 0.5 * (acts[0] + acts[1])
    for r in (0, 1):
        hsT_ref[r, :, :] = h1 * n[2 * r:2 * r + 1, :]


def _tc_combine1(agg1, xT, deg, w1sT, b1sT):
    g = NPAD // _R
    return pl.pallas_call(
        _combine1_body,
        grid=(g,),
        in_specs=[pl.BlockSpec((2, D, _R), lambda i: (0, 0, i)),
                  pl.BlockSpec((D, _R), lambda i: (0, i)),
                  pl.BlockSpec((4, _R), lambda i: (0, i)),
                  pl.BlockSpec((2, D, D), lambda i: (0, 0, 0)),
                  pl.BlockSpec((2, D, 1), lambda i: (0, 0, 0))],
        out_specs=pl.BlockSpec((2, D, _R), lambda i: (0, 0, i)),
        out_shape=jax.ShapeDtypeStruct((2, D, NPAD), jnp.float32),
    )(agg1, xT, deg, w1sT, b1sT)


def _combine2_body(agg_ref, xT_ref, deg_ref, w_ref, b_ref, wl_ref, bl_ref,
                   outT_ref):
    n = _norms(deg_ref[...])
    xb = xT_ref[...]
    ts = []
    for r in (0, 1):
        a = agg_ref[r] * n[2 * r + 1:2 * r + 2, :]
        f = (1.0 - ALPHA) * a + ALPHA * xb
        ts.append((1.0 - BETA2) * f
                  + BETA2 * jnp.dot(w_ref[r], f,
                                    preferred_element_type=jnp.float32)
                  + b_ref[r])
    h2 = 0.5 * (ts[0] + ts[1])
    outT_ref[...] = (jnp.dot(wl_ref[...], h2,
                             preferred_element_type=jnp.float32)
                     + bl_ref[...])


def _tc_combine2(agg2, xT, deg, w2sT, b2sT, wlin, blT):
    g = NPAD // _R
    return pl.pallas_call(
        _combine2_body,
        grid=(g,),
        in_specs=[pl.BlockSpec((2, D, _R), lambda i: (0, 0, i)),
                  pl.BlockSpec((D, _R), lambda i: (0, i)),
                  pl.BlockSpec((4, _R), lambda i: (0, i)),
                  pl.BlockSpec((2, D, D), lambda i: (0, 0, 0)),
                  pl.BlockSpec((2, D, 1), lambda i: (0, 0, 0)),
                  pl.BlockSpec((D, D), lambda i: (0, 0)),
                  pl.BlockSpec((D, 1), lambda i: (0, 0))],
        out_specs=pl.BlockSpec((D, _R), lambda i: (0, i)),
        out_shape=jax.ShapeDtypeStruct((D, NPAD), jnp.float32),
    )(agg2, xT, deg, w2sT, b2sT, wlin, blT)


def kernel(x, edge_index_rel0, edge_index_rel1,
           W1_rel0, W1_rel1, W2_rel0, W2_rel1,
           b1_rel0, b1_rel1, b2_rel0, b2_rel1,
           W_lin, b_lin):
    x_pad = jnp.pad(x, ((0, NPAD - N), (0, 0)))
    xT = x_pad.T
    ei = jnp.stack([edge_index_rel0, edge_index_rel1]).astype(jnp.int32)
    ei_pad = jnp.pad(ei, ((0, 0), (0, 0), (0, EPAD - E)), constant_values=N)
    hidx = ei_pad.reshape(2, 2, EPAD // 128, 128)

    deg = _sc_hist(hidx)
    xsT = _tc_scale(deg, xT)
    agg1 = _sc_agg(xsT, ei_pad)
    w1sT = jnp.stack([W1_rel0.T, W1_rel1.T])
    b1sT = jnp.stack([b1_rel0, b1_rel1])[:, :, None]
    hsT = _tc_combine1(agg1, xT, deg, w1sT, b1sT)
    agg2 = _sc_agg(hsT, ei_pad)
    w2sT = jnp.stack([W2_rel0.T, W2_rel1.T])
    b2sT = jnp.stack([b2_rel0, b2_rel1])[:, :, None]
    outT = _tc_combine2(agg2, xT, deg, w2sT, b2sT, W_lin, b_lin[:, None])
    return outT.T[:N]


# speedup vs baseline: 1.0874x; 1.0874x over previous
import functools
import math

import jax
import jax.numpy as jnp
from jax import lax
from jax.experimental import pallas as pl
from jax.experimental.pallas import tpu as pltpu
from jax.experimental.pallas import tpu_sc as plsc

ALPHA = 0.5
BETA1 = math.log(2.0)
BETA2 = math.log(1.5)

NC = 2
NS = 16

N = 10000
D = 128
E = 160000
NPAD = 10240
EPAD = 163840
CHK = 8192
NCHK = EPAD // CHK
RPT = NPAD // NS
CPT = EPAD // NS // 128
FPT = 4

_mesh = plsc.VectorSubcoreMesh(
    core_axis_name="c", subcore_axis_name="s", num_cores=NC, num_subcores=NS)


@functools.partial(
    pl.kernel,
    out_type=jax.ShapeDtypeStruct((4, NPAD), jnp.float32),
    mesh=_mesh,
    scratch_types=[
        pltpu.VMEM((8, 128), jnp.int32),
        pltpu.VMEM((128,), jnp.float32),
        pltpu.VMEM((RPT,), jnp.float32),
        pltpu.VMEM_SHARED((NPAD,), jnp.float32),
        pltpu.VMEM_SHARED((NPAD,), jnp.float32),
    ],
)
def _sc_hist(hidx, deg_out, idxb, ones, zb, hist_s, hist_d):
    c = lax.axis_index("c")
    s = lax.axis_index("s")

    ov = jnp.ones((16,), jnp.float32)
    zv = jnp.zeros((16,), jnp.float32)

    def fill_ones(i, carry):
        ones[pl.ds(i * 16, 16)] = ov
        return carry
    lax.fori_loop(0, 8, fill_ones, 0)

    def fill_zb(i, carry):
        zb[pl.ds(i * 16, 16)] = zv
        return carry
    lax.fori_loop(0, RPT // 16, fill_zb, 0)

    pltpu.sync_copy(zb, hist_s.at[pl.ds(s * RPT, RPT)])
    pltpu.sync_copy(zb, hist_d.at[pl.ds(s * RPT, RPT)])
    plsc.subcore_barrier()

    base = s * CPT
    for j, hist in ((0, hist_s), (1, hist_d)):
        def grp(g, carry):
            pltpu.sync_copy(hidx.at[c, j, pl.ds(base + g * 8, 8), :], idxb)
            for k in range(8):
                pltpu.sync_copy(ones, hist.at[idxb.at[k]], add=True)
            return carry
        lax.fori_loop(0, CPT // 8, grp, 0)

    plsc.subcore_barrier()
    pltpu.sync_copy(hist_s.at[pl.ds(s * RPT, RPT)],
                    deg_out.at[2 * c, pl.ds(s * RPT, RPT)])
    pltpu.sync_copy(hist_d.at[pl.ds(s * RPT, RPT)],
                    deg_out.at[2 * c + 1, pl.ds(s * RPT, RPT)])


@functools.partial(
    pl.kernel,
    out_type=jax.ShapeDtypeStruct((2, D, NPAD), jnp.float32),
    mesh=_mesh,
    compiler_params=pltpu.CompilerParams(needs_layout_passes=False),
    scratch_types=[
        pltpu.VMEM((2, CHK), jnp.int32),
        pltpu.VMEM((2, CHK), jnp.int32),
        pltpu.VMEM((FPT * NPAD,), jnp.float32),
        pltpu.VMEM((FPT * NPAD,), jnp.float32),
        pltpu.SemaphoreType.DMA,
        pltpu.SemaphoreType.DMA,
    ],
)
def _sc_agg(tblT, eflat, out, srcb, dstb, tbl, acc, sem0, sem1):
    c = lax.axis_index("c")
    s = lax.axis_index("s")
    sems = (sem0, sem1)

    zv = jnp.zeros((16,), jnp.float32)

    def fire(pi, g):
        pltpu.async_copy(eflat.at[c, 0, pl.ds(g * CHK, CHK)],
                         srcb.at[pi], sems[pi])
        pltpu.async_copy(eflat.at[c, 1, pl.ds(g * CHK, CHK)],
                         dstb.at[pi], sems[pi])

    def drain(pi, g):
        pltpu.make_async_copy(eflat.at[c, 0, pl.ds(g * CHK, CHK)],
                              srcb.at[pi], sems[pi]).wait()
        pltpu.make_async_copy(eflat.at[c, 1, pl.ds(g * CHK, CHK)],
                              dstb.at[pi], sems[pi]).wait()

    def process(pi):
        def ed(i, carry2):
            sv = [srcb[pi, pl.ds(i * 64 + q * 16, 16)] for q in range(4)]
            dv = [dstb[pi, pl.ds(i * 64 + q * 16, 16)] for q in range(4)]
            vs = [[plsc.load_gather(tbl, [sv[q] + (j * NPAD)])
                   for j in range(FPT)] for q in range(4)]
            for q in range(4):
                for j in range(FPT):
                    plsc.addupdate_scatter(acc, [dv[q] + (j * NPAD)],
                                           vs[q][j])
            return carry2
        lax.fori_loop(0, CHK // 64, ed, 0)

    for p in (0, 1):
        row0 = p * (NS * FPT) + s * FPT
        for j in range(FPT):
            pltpu.sync_copy(tblT.at[c, row0 + j, :],
                            tbl.at[pl.ds(j * NPAD, NPAD)])

        fire(0, 0)

        def zrow(i, carry):
            acc[pl.ds(i * 16, 16)] = zv
            return carry
        lax.fori_loop(0, FPT * NPAD // 16, zrow, 0)

        def pair(k, carry):
            fire(1, 2 * k + 1)
            drain(0, 2 * k)
            process(0)

            @pl.when(k < NCHK // 2 - 1)
            def _():
                fire(0, 2 * k + 2)

            drain(1, 2 * k + 1)
            process(1)
            return carry
        lax.fori_loop(0, NCHK // 2, pair, 0)

        for j in range(FPT):
            pltpu.sync_copy(acc.at[pl.ds(j * NPAD, NPAD)],
                            out.at[c, row0 + j, :])


_R = 1024


def _norms(deg_blk):
    return lax.rsqrt(jnp.maximum(deg_blk, 1.0))


def _scale_body(deg_ref, xT_ref, xsT_ref):
    n = _norms(deg_ref[...])
    xb = xT_ref[...]
    for r in (0, 1):
        xsT_ref[r, :, :] = xb * n[2 * r:2 * r + 1, :]


def _tc_scale(deg, xT):
    g = NPAD // _R
    return pl.pallas_call(
        _scale_body,
        grid=(g,),
        in_specs=[pl.BlockSpec((4, _R), lambda i: (0, i)),
                  pl.BlockSpec((D, _R), lambda i: (0, i))],
        out_specs=pl.BlockSpec((2, D, _R), lambda i: (0, 0, i)),
        out_shape=jax.ShapeDtypeStruct((2, D, NPAD), jnp.float32),
    )(deg, xT)


def _combine1_body(agg_ref, xT_ref, deg_ref, w_ref, b_ref, hsT_ref):
    n = _norms(deg_ref[...])
    xb = xT_ref[...]
    acts = []
    for r in (0, 1):
        a = agg_ref[r] * n[2 * r + 1:2 * r + 2, :]
        f = (1.0 - ALPHA) * a + ALPHA * xb
        t = ((1.0 - BETA1) * f
             + BETA1 * jnp.dot(w_ref[r], f, preferred_element_type=jnp.float32)
             + b_ref[r])
        acts.append(jnp.where(t >= 0, t, 0.01 * t))
    h1 = 0.5 * (acts[0] + acts[1])
    for r in (0, 1):
        hsT_ref[r, :, :] = h1 * n[2 * r:2 * r + 1, :]


def _tc_combine1(agg1, xT, deg, w1sT, b1sT):
    g = NPAD // _R
    return pl.pallas_call(
        _combine1_body,
        grid=(g,),
        in_specs=[pl.BlockSpec((2, D, _R), lambda i: (0, 0, i)),
                  pl.BlockSpec((D, _R), lambda i: (0, i)),
                  pl.BlockSpec((4, _R), lambda i: (0, i)),
                  pl.BlockSpec((2, D, D), lambda i: (0, 0, 0)),
                  pl.BlockSpec((2, D, 1), lambda i: (0, 0, 0))],
        out_specs=pl.BlockSpec((2, D, _R), lambda i: (0, 0, i)),
        out_shape=jax.ShapeDtypeStruct((2, D, NPAD), jnp.float32),
    )(agg1, xT, deg, w1sT, b1sT)


def _combine2_body(agg_ref, xT_ref, deg_ref, w_ref, b_ref, wl_ref, bl_ref,
                   outT_ref):
    n = _norms(deg_ref[...])
    xb = xT_ref[...]
    ts = []
    for r in (0, 1):
        a = agg_ref[r] * n[2 * r + 1:2 * r + 2, :]
        f = (1.0 - ALPHA) * a + ALPHA * xb
        ts.append((1.0 - BETA2) * f
                  + BETA2 * jnp.dot(w_ref[r], f,
                                    preferred_element_type=jnp.float32)
                  + b_ref[r])
    h2 = 0.5 * (ts[0] + ts[1])
    outT_ref[...] = (jnp.dot(wl_ref[...], h2,
                             preferred_element_type=jnp.float32)
                     + bl_ref[...])


def _tc_combine2(agg2, xT, deg, w2sT, b2sT, wlin, blT):
    g = NPAD // _R
    return pl.pallas_call(
        _combine2_body,
        grid=(g,),
        in_specs=[pl.BlockSpec((2, D, _R), lambda i: (0, 0, i)),
                  pl.BlockSpec((D, _R), lambda i: (0, i)),
                  pl.BlockSpec((4, _R), lambda i: (0, i)),
                  pl.BlockSpec((2, D, D), lambda i: (0, 0, 0)),
                  pl.BlockSpec((2, D, 1), lambda i: (0, 0, 0)),
                  pl.BlockSpec((D, D), lambda i: (0, 0)),
                  pl.BlockSpec((D, 1), lambda i: (0, 0))],
        out_specs=pl.BlockSpec((D, _R), lambda i: (0, i)),
        out_shape=jax.ShapeDtypeStruct((D, NPAD), jnp.float32),
    )(agg2, xT, deg, w2sT, b2sT, wlin, blT)


def kernel(x, edge_index_rel0, edge_index_rel1,
           W1_rel0, W1_rel1, W2_rel0, W2_rel1,
           b1_rel0, b1_rel1, b2_rel0, b2_rel1,
           W_lin, b_lin):
    x_pad = jnp.pad(x, ((0, NPAD - N), (0, 0)))
    xT = x_pad.T
    ei = jnp.stack([edge_index_rel0, edge_index_rel1]).astype(jnp.int32)
    ei_pad = jnp.pad(ei, ((0, 0), (0, 0), (0, EPAD - E)), constant_values=N)
    hidx = ei_pad.reshape(2, 2, EPAD // 128, 128)

    deg = _sc_hist(hidx)
    xsT = _tc_scale(deg, xT)
    agg1 = _sc_agg(xsT, ei_pad)
    w1sT = jnp.stack([W1_rel0.T, W1_rel1.T])
    b1sT = jnp.stack([b1_rel0, b1_rel1])[:, :, None]
    hsT = _tc_combine1(agg1, xT, deg, w1sT, b1sT)
    agg2 = _sc_agg(hsT, ei_pad)
    w2sT = jnp.stack([W2_rel0.T, W2_rel1.T])
    b2sT = jnp.stack([b2_rel0, b2_rel1])[:, :, None]
    outT = _tc_combine2(agg2, xT, deg, w2sT, b2sT, W_lin, b_lin[:, None])
    return outT.T[:N]
